# baseline (speedup 1.0000x reference)
import numpy as np
import jax
import jax.numpy as jnp
from jax import lax
from jax.experimental import pallas as pl
from jax.experimental.pallas import tpu as pltpu
from jax.experimental.pallas import tpu_sc as plsc

N = 26
NROW = N * N
D = 16
ROWLEN = NROW * D
NPAIR = N * (N + 1) // 2
NPAD = 352
NG = NPAD // 16
BATCH = 4096
NC, NS = 2, 16
NW = NC * NS
BPW = BATCH // NW


def _pair_elem_indices():
    xs, ys = [], []
    for i in range(N):
        for j in range(i, N):
            xs.append((i * N + j) * D)
            ys.append((j * N + i) * D)
    while len(xs) < NPAD:
        xs.append(0)
        ys.append(0)
    return np.asarray(xs, np.int32), np.asarray(ys, np.int32)


_XE, _YE = _pair_elem_indices()


def _body(inp_hbm, xe_hbm, ye_hbm, out_hbm,
          xe_v, ye_v, buf0, buf1, out_v, sem0, sem1):
    c = lax.axis_index("c")
    s = lax.axis_index("s")
    wid = s * NC + c
    base = wid * BPW

    pltpu.sync_copy(xe_hbm, xe_v)
    pltpu.sync_copy(ye_hbm, ye_v)

    bufs = (buf0, buf1)
    sems = (sem0, sem1)
    pltpu.async_copy(inp_hbm.at[base], buf0, sem0)
    pltpu.async_copy(inp_hbm.at[base + 1], buf1, sem1)

    @pl.loop(0, BPW, step=2)
    def _batches(i):
        for off in range(2):
            b = i + off
            buf = bufs[off]
            sem = sems[off]
            pltpu.make_async_copy(inp_hbm.at[base + b], buf, sem).wait()

            @pl.loop(0, NG)
            def _groups(g):
                xg = xe_v[pl.ds(g * 16, 16)]
                yg = ye_v[pl.ds(g * 16, 16)]
                acc = plsc.load_gather(buf, [xg]) * plsc.load_gather(buf, [yg])
                for d in range(1, D):
                    xv = plsc.load_gather(buf, [xg + d])
                    yv = plsc.load_gather(buf, [yg + d])
                    acc = acc + xv * yv
                out_v[b, pl.ds(g * 16, 16)] = acc

            @pl.when(b + 2 < BPW)
            def _prefetch():
                pltpu.async_copy(inp_hbm.at[base + b + 2], buf, sem)

    pltpu.sync_copy(out_v, out_hbm.at[pl.ds(base, BPW)])


@jax.jit
def kernel(input):
    inp = input.reshape(BATCH, ROWLEN)
    xe = jnp.asarray(_XE)
    ye = jnp.asarray(_YE)
    mesh = plsc.VectorSubcoreMesh(
        core_axis_name="c", subcore_axis_name="s",
        num_cores=NC, num_subcores=NS)
    f = pl.kernel(
        _body,
        out_type=jax.ShapeDtypeStruct((BATCH, NPAD), jnp.float32),
        mesh=mesh,
        compiler_params=pltpu.CompilerParams(needs_layout_passes=False),
        scratch_types=[
            pltpu.VMEM((NPAD,), jnp.int32),
            pltpu.VMEM((NPAD,), jnp.int32),
            pltpu.VMEM((ROWLEN,), jnp.float32),
            pltpu.VMEM((ROWLEN,), jnp.float32),
            pltpu.VMEM((BPW, NPAD), jnp.float32),
            pltpu.SemaphoreType.DMA,
            pltpu.SemaphoreType.DMA,
        ],
    )
    out = f(inp, xe, ye)
    return out[:, :NPAIR]


# speedup vs baseline: 1.8141x; 1.8141x over previous
import numpy as np
import jax
import jax.numpy as jnp
from jax import lax
from jax.experimental import pallas as pl
from jax.experimental.pallas import tpu as pltpu
from jax.experimental.pallas import tpu_sc as plsc

N = 26
NROW = N * N
D = 16
ROWLEN = NROW * D
NPAIR = N * (N + 1) // 2
NPAD = 352
BATCH = 4096
NC, NS = 2, 16
NW = NC * NS
BPW = BATCH // NW


def _pair_offsets():
    xs, ys = [], []
    for i in range(N):
        for j in range(i, N):
            xs.append((i * N + j) * D)
            ys.append((j * N + i) * D)
    while len(xs) < NPAD:
        xs.append(0)
        ys.append(0)
    return np.asarray(xs, np.int32), np.asarray(ys, np.int32)


_XE, _YE = _pair_offsets()
_REV = np.asarray([int(f"{l:04b}"[::-1], 2) for l in range(16)], np.int32)


def _body(inp_hbm, xe_hbm, ye_hbm, out_hbm,
          xe_v, ye_v, buf0, buf1, out_v, sem0, sem1):
    c = lax.axis_index("c")
    s = lax.axis_index("s")
    wid = s * NC + c
    base = wid * BPW

    pltpu.sync_copy(xe_hbm, xe_v)
    pltpu.sync_copy(ye_hbm, ye_v)

    lane = lax.broadcasted_iota(jnp.int32, (D,), 0)
    perms = [lane ^ h for h in (8, 4, 2, 1)]
    masks = [(lane & h) == 0 for h in (8, 4, 2, 1)]
    rev_v = (((lane & 1) << 3) | ((lane & 2) << 1)
             | ((lane & 4) >> 1) | ((lane & 8) >> 3))

    bufs = (buf0, buf1)
    sems = (sem0, sem1)
    pltpu.async_copy(inp_hbm.at[base], buf0, sem0)
    pltpu.async_copy(inp_hbm.at[base + 1], buf1, sem1)

    @pl.loop(0, BPW, step=2)
    def _batches(i):
        for off in range(2):
            b = i + off
            buf = bufs[off]
            sem = sems[off]
            pltpu.make_async_copy(inp_hbm.at[base + b], buf, sem).wait()
            obase = b * NPAD

            @pl.loop(0, NPAD // D)
            def _groups(g):
                xoffs = xe_v[pl.ds(g * D, D)]
                yoffs = ye_v[pl.ds(g * D, D)]
                kbase = obase + g * D
                vecs = []
                for t in range(D):
                    xv = buf[pl.ds(xoffs[t], D)]
                    yv = buf[pl.ds(yoffs[t], D)]
                    vecs.append(xv * yv)
                for perm, m in zip(perms, masks):
                    nxt = []
                    for a in range(0, len(vecs), 2):
                        hi = jnp.where(m, vecs[a], vecs[a + 1])
                        lo = jnp.where(m, vecs[a + 1], vecs[a])
                        nxt.append(hi + jnp.take_along_axis(lo, perm, axis=0))
                    vecs = nxt
                idx = jnp.broadcast_to(kbase, (D,)).astype(jnp.int32) + rev_v
                plsc.store_scatter(out_v, [idx], vecs[0])

            @pl.when(b + 2 < BPW)
            def _prefetch():
                pltpu.async_copy(inp_hbm.at[base + b + 2], buf, sem)

    pltpu.sync_copy(out_v, out_hbm.at[pl.ds(base * NPAD, BPW * NPAD)])


@jax.jit
def kernel(input):
    inp = input.reshape(BATCH, ROWLEN)
    xe = jnp.asarray(_XE)
    ye = jnp.asarray(_YE)
    mesh = plsc.VectorSubcoreMesh(
        core_axis_name="c", subcore_axis_name="s",
        num_cores=NC, num_subcores=NS)
    f = pl.kernel(
        _body,
        out_type=jax.ShapeDtypeStruct((BATCH * NPAD,), jnp.float32),
        mesh=mesh,
        compiler_params=pltpu.CompilerParams(needs_layout_passes=False),
        scratch_types=[
            pltpu.VMEM((NPAD,), jnp.int32),
            pltpu.VMEM((NPAD,), jnp.int32),
            pltpu.VMEM((ROWLEN,), jnp.float32),
            pltpu.VMEM((ROWLEN,), jnp.float32),
            pltpu.VMEM((BPW * NPAD,), jnp.float32),
            pltpu.SemaphoreType.DMA,
            pltpu.SemaphoreType.DMA,
        ],
    )
    out = f(inp, xe, ye)
    return out.reshape(BATCH, NPAD)[:, :NPAIR]


# speedup vs baseline: 4.2863x; 2.3628x over previous
import numpy as np
import jax
import jax.numpy as jnp
from jax import lax
from jax.experimental import pallas as pl
from jax.experimental.pallas import tpu as pltpu
from jax.experimental.pallas import tpu_sc as plsc

N = 26
D = 16
BATCH = 4096
NPAIR = N * (N + 1) // 2
NC, NS = 2, 16
NW = NC * NS
BCOL = BATCH // NW
LG = BCOL // 16

_OUT_ROWS = np.asarray(
    [i * 32 + j for i in range(N) for j in range(i, N)], np.int32)


def _body(inp, out, xb0, xb1, yb0, yb1, acc, sems):
    c = lax.axis_index("c")
    s = lax.axis_index("s")
    b0 = (s * NC + c) * BCOL

    xbufs = (xb0, xb1)
    ybufs = (yb0, yb1)

    def slabs(i, h):
        r = N - i
        src_x = inp.at[i, pl.ds(i, r), pl.ds(h * 8, 8), pl.ds(b0, BCOL)]
        src_y = inp.at[pl.ds(i, r), i, pl.ds(h * 8, 8), pl.ds(b0, BCOL)]
        return src_x, src_y

    def start(i, h):
        src_x, src_y = slabs(i, h)
        r = N - i
        pltpu.async_copy(src_x, xbufs[h].at[pl.ds(0, r)], sems[h])
        pltpu.async_copy(src_y, ybufs[h].at[pl.ds(0, r)], sems[h])

    def wait(i, h):
        src_x, src_y = slabs(i, h)
        r = N - i
        pltpu.make_async_copy(src_x, xbufs[h].at[pl.ds(0, r)], sems[h]).wait()
        pltpu.make_async_copy(src_y, ybufs[h].at[pl.ds(0, r)], sems[h]).wait()

    start(0, 0)
    start(0, 1)

    for i in range(N):
        r = N - i
        for h in range(2):
            wait(i, h)
            xb = xbufs[h]
            yb = ybufs[h]

            if h == 0:
                @pl.loop(0, r)
                def _pairs0(j):
                    @pl.loop(0, LG, unroll=2)
                    def _lanes0(lg):
                        bs = lg * 16
                        v = xb[j, 0, pl.ds(bs, 16)] * yb[j, 0, pl.ds(bs, 16)]
                        for dd in range(1, 8):
                            v += xb[j, dd, pl.ds(bs, 16)] * yb[j, dd, pl.ds(bs, 16)]
                        acc[i + j, pl.ds(bs, 16)] = v
            else:
                @pl.loop(0, r)
                def _pairs1(j):
                    @pl.loop(0, LG, unroll=2)
                    def _lanes1(lg):
                        bs = lg * 16
                        v = acc[i + j, pl.ds(bs, 16)]
                        for dd in range(8):
                            v += xb[j, dd, pl.ds(bs, 16)] * yb[j, dd, pl.ds(bs, 16)]
                        acc[i + j, pl.ds(bs, 16)] = v

            if i + 1 < N:
                start(i + 1, h)

        pltpu.sync_copy(acc, out.at[i, pl.ds(0, 32), pl.ds(b0, BCOL)])


@jax.jit
def kernel(input):
    inp = jnp.transpose(input, (1, 2, 0)).reshape(N, N, D, BATCH)
    mesh = plsc.VectorSubcoreMesh(
        core_axis_name="c", subcore_axis_name="s",
        num_cores=NC, num_subcores=NS)
    f = pl.kernel(
        _body,
        out_type=jax.ShapeDtypeStruct((N, 32, BATCH), jnp.float32),
        mesh=mesh,
        compiler_params=pltpu.CompilerParams(needs_layout_passes=False),
        scratch_types=[
            pltpu.VMEM((N, 8, BCOL), jnp.float32),
            pltpu.VMEM((N, 8, BCOL), jnp.float32),
            pltpu.VMEM((N, 8, BCOL), jnp.float32),
            pltpu.VMEM((N, 8, BCOL), jnp.float32),
            pltpu.VMEM((32, BCOL), jnp.float32),
            (pltpu.SemaphoreType.DMA, pltpu.SemaphoreType.DMA),
        ],
    )
    out = f(inp)
    rows = jnp.asarray(_OUT_ROWS)
    return jnp.take(out.reshape(N * 32, BATCH), rows, axis=0).T


# speedup vs baseline: 5.3930x; 1.2582x over previous
import numpy as np
import jax
import jax.numpy as jnp
from jax import lax
from jax.experimental import pallas as pl
from jax.experimental.pallas import tpu as pltpu
from jax.experimental.pallas import tpu_sc as plsc

N = 26
D = 16
BATCH = 4096
NPAIR = N * (N + 1) // 2
NC, NS = 2, 16
NW = NC * NS
BCOL = BATCH // NW
LG = BCOL // 16

IT = 6
BCT = 2048

_TC_ROWS = np.asarray(
    [i * N + j for i in range(IT) for j in range(i, N)], np.int32)
_SC_ROWS = np.asarray(
    [i * 32 + j for i in range(IT, N) for j in range(i, N)], np.int32)


def _body(inp, out, xb0, xb1, yb0, yb1, acc, sems):
    c = lax.axis_index("c")
    s = lax.axis_index("s")
    b0 = (s * NC + c) * BCOL

    xbufs = (xb0, xb1)
    ybufs = (yb0, yb1)

    def slabs(i, h):
        r = N - i
        src_x = inp.at[i, pl.ds(i, r), pl.ds(h * 8, 8), pl.ds(b0, BCOL)]
        src_y = inp.at[pl.ds(i, r), i, pl.ds(h * 8, 8), pl.ds(b0, BCOL)]
        return src_x, src_y

    def start(i, h):
        src_x, src_y = slabs(i, h)
        r = N - i
        pltpu.async_copy(src_x, xbufs[h].at[pl.ds(0, r)], sems[h])
        pltpu.async_copy(src_y, ybufs[h].at[pl.ds(0, r)], sems[h])

    def wait(i, h):
        src_x, src_y = slabs(i, h)
        r = N - i
        pltpu.make_async_copy(src_x, xbufs[h].at[pl.ds(0, r)], sems[h]).wait()
        pltpu.make_async_copy(src_y, ybufs[h].at[pl.ds(0, r)], sems[h]).wait()

    start(IT, 0)
    start(IT, 1)

    for i in range(IT, N):
        r = N - i
        for h in range(2):
            wait(i, h)
            xb = xbufs[h]
            yb = ybufs[h]

            if h == 0:
                @pl.loop(0, r)
                def _pairs0(j):
                    @pl.loop(0, LG, unroll=2)
                    def _lanes0(lg):
                        bs = lg * 16
                        v = xb[j, 0, pl.ds(bs, 16)] * yb[j, 0, pl.ds(bs, 16)]
                        for dd in range(1, 8):
                            v += xb[j, dd, pl.ds(bs, 16)] * yb[j, dd, pl.ds(bs, 16)]
                        acc[i + j, pl.ds(bs, 16)] = v
            else:
                @pl.loop(0, r)
                def _pairs1(j):
                    @pl.loop(0, LG, unroll=2)
                    def _lanes1(lg):
                        bs = lg * 16
                        v = acc[i + j, pl.ds(bs, 16)]
                        for dd in range(8):
                            v += xb[j, dd, pl.ds(bs, 16)] * yb[j, dd, pl.ds(bs, 16)]
                        acc[i + j, pl.ds(bs, 16)] = v

            if i + 1 < N:
                start(i + 1, h)

        pltpu.sync_copy(acc, out.at[i, pl.ds(0, 32), pl.ds(b0, BCOL)])


def _tc_body(x_ref, y_ref, o_ref):
    o_ref[0] = jnp.sum(x_ref[0] * y_ref[:, 0], axis=1)


@jax.jit
def kernel(input):
    inp = jnp.transpose(input, (1, 2, 0)).reshape(N, N, D, BATCH)
    mesh = plsc.VectorSubcoreMesh(
        core_axis_name="c", subcore_axis_name="s",
        num_cores=NC, num_subcores=NS)
    f = pl.kernel(
        _body,
        out_type=jax.ShapeDtypeStruct((N, 32, BATCH), jnp.float32),
        mesh=mesh,
        compiler_params=pltpu.CompilerParams(needs_layout_passes=False),
        scratch_types=[
            pltpu.VMEM((N, 8, BCOL), jnp.float32),
            pltpu.VMEM((N, 8, BCOL), jnp.float32),
            pltpu.VMEM((N, 8, BCOL), jnp.float32),
            pltpu.VMEM((N, 8, BCOL), jnp.float32),
            pltpu.VMEM((32, BCOL), jnp.float32),
            (pltpu.SemaphoreType.DMA, pltpu.SemaphoreType.DMA),
        ],
    )
    sc_out = f(inp)
    tc = pl.pallas_call(
        _tc_body,
        grid=(IT, BATCH // BCT),
        in_specs=[
            pl.BlockSpec((1, N, D, BCT), lambda ib, cb: (ib, 0, 0, cb)),
            pl.BlockSpec((N, 1, D, BCT), lambda ib, cb: (0, ib, 0, cb)),
        ],
        out_specs=pl.BlockSpec((1, N, BCT), lambda ib, cb: (ib, 0, cb)),
        out_shape=jax.ShapeDtypeStruct((IT, N, BATCH), jnp.float32),
    )
    tc_out = tc(inp, inp)
    a = jnp.take(tc_out.reshape(IT * N, BATCH), jnp.asarray(_TC_ROWS), axis=0)
    b = jnp.take(sc_out.reshape(N * 32, BATCH), jnp.asarray(_SC_ROWS), axis=0)
    return jnp.concatenate([a, b], axis=0).T


# speedup vs baseline: 6.4816x; 1.2019x over previous
import numpy as np
import jax
import jax.numpy as jnp
from jax import lax
from jax.experimental import pallas as pl
from jax.experimental.pallas import tpu as pltpu
from jax.experimental.pallas import tpu_sc as plsc

N = 26
D = 16
BATCH = 4096
NPAIR = N * (N + 1) // 2
NC, NS = 2, 16
NW = NC * NS
BCOL = BATCH // NW
LG = BCOL // 16

IT = 9
NSC = sum(N - i for i in range(IT, N))
NSCP = 160
_KB = {}
_k = 0
for _i in range(IT, N):
    _KB[_i] = _k
    _k += N - _i
BCT = 2048

_TC_ROWS = np.asarray(
    [i * N + j for i in range(IT) for j in range(i, N)], np.int32)


def _body(inp, out, xb0, xb1, yb0, yb1, out_v, sems):
    c = lax.axis_index("c")
    s = lax.axis_index("s")
    b0 = (s * NC + c) * BCOL

    xbufs = (xb0, xb1)
    ybufs = (yb0, yb1)

    def slabs(i, h):
        r = N - i
        src_x = inp.at[i, pl.ds(i, r), pl.ds(h * 8, 8), pl.ds(b0, BCOL)]
        src_y = inp.at[pl.ds(i, r), i, pl.ds(h * 8, 8), pl.ds(b0, BCOL)]
        return src_x, src_y

    def start(i, h):
        src_x, src_y = slabs(i, h)
        r = N - i
        pltpu.async_copy(src_x, xbufs[h].at[pl.ds(0, r)], sems[h])
        pltpu.async_copy(src_y, ybufs[h].at[pl.ds(0, r)], sems[h])

    def wait(i, h):
        src_x, src_y = slabs(i, h)
        r = N - i
        pltpu.make_async_copy(src_x, xbufs[h].at[pl.ds(0, r)], sems[h]).wait()
        pltpu.make_async_copy(src_y, ybufs[h].at[pl.ds(0, r)], sems[h]).wait()

    start(IT, 0)
    start(IT, 1)

    for i in range(IT, N):
        r = N - i
        for h in range(2):
            wait(i, h)
            xb = xbufs[h]
            yb = ybufs[h]

            if h == 0:
                @pl.loop(0, r)
                def _pairs0(j):
                    @pl.loop(0, LG, unroll=2)
                    def _lanes0(lg):
                        bs = lg * 16
                        v = xb[j, 0, pl.ds(bs, 16)] * yb[j, 0, pl.ds(bs, 16)]
                        for dd in range(1, 8):
                            v += xb[j, dd, pl.ds(bs, 16)] * yb[j, dd, pl.ds(bs, 16)]
                        out_v[_KB[i] + j, pl.ds(bs, 16)] = v
            else:
                @pl.loop(0, r)
                def _pairs1(j):
                    @pl.loop(0, LG, unroll=2)
                    def _lanes1(lg):
                        bs = lg * 16
                        v = out_v[_KB[i] + j, pl.ds(bs, 16)]
                        for dd in range(8):
                            v += xb[j, dd, pl.ds(bs, 16)] * yb[j, dd, pl.ds(bs, 16)]
                        out_v[_KB[i] + j, pl.ds(bs, 16)] = v

            if i + 1 < N:
                start(i + 1, h)

    pltpu.sync_copy(out_v, out.at[pl.ds(0, NSCP), pl.ds(b0, BCOL)])


def _tc_body(x_ref, y_ref, o_ref):
    o_ref[0] = jnp.sum(x_ref[0] * y_ref[:, 0], axis=1)


@jax.jit
def kernel(input):
    inp = jnp.transpose(input, (1, 2, 0)).reshape(N, N, D, BATCH)
    mesh = plsc.VectorSubcoreMesh(
        core_axis_name="c", subcore_axis_name="s",
        num_cores=NC, num_subcores=NS)
    f = pl.kernel(
        _body,
        out_type=jax.ShapeDtypeStruct((NSCP, BATCH), jnp.float32),
        mesh=mesh,
        compiler_params=pltpu.CompilerParams(needs_layout_passes=False),
        scratch_types=[
            pltpu.VMEM((N - IT, 8, BCOL), jnp.float32),
            pltpu.VMEM((N - IT, 8, BCOL), jnp.float32),
            pltpu.VMEM((N - IT, 8, BCOL), jnp.float32),
            pltpu.VMEM((N - IT, 8, BCOL), jnp.float32),
            pltpu.VMEM((NSCP, BCOL), jnp.float32),
            (pltpu.SemaphoreType.DMA, pltpu.SemaphoreType.DMA),
        ],
    )
    sc_out = f(inp)
    tc = pl.pallas_call(
        _tc_body,
        grid=(IT, BATCH // BCT),
        in_specs=[
            pl.BlockSpec((1, N, D, BCT), lambda ib, cb: (ib, 0, 0, cb)),
            pl.BlockSpec((N, 1, D, BCT), lambda ib, cb: (0, ib, 0, cb)),
        ],
        out_specs=pl.BlockSpec((1, N, BCT), lambda ib, cb: (ib, 0, cb)),
        out_shape=jax.ShapeDtypeStruct((IT, N, BATCH), jnp.float32),
    )
    tc_out = tc(inp, inp)
    a = jnp.take(tc_out.reshape(IT * N, BATCH), jnp.asarray(_TC_ROWS), axis=0)
    return jnp.concatenate([a, sc_out[:NSC]], axis=0).T


# speedup vs baseline: 6.5585x; 1.0119x over previous
import numpy as np
import jax
import jax.numpy as jnp
from jax import lax
from jax.experimental import pallas as pl
from jax.experimental.pallas import tpu as pltpu
from jax.experimental.pallas import tpu_sc as plsc

N = 26
D = 16
BATCH = 4096
NPAIR = N * (N + 1) // 2
NC, NS = 2, 16
NW = NC * NS
BCOL = BATCH // NW
LG = BCOL // 16

IT = 9
NSC = sum(N - i for i in range(IT, N))
NSCP = 160
_KB = {}
_k = 0
for _i in range(IT, N):
    _KB[_i] = _k
    _k += N - _i
BCT = 2048

_TC_ROWS = np.asarray(
    [i * 32 + j for i in range(IT) for j in range(i, N)], np.int32)


def _body(inp, out, xb0, xb1, yb0, yb1, out_v, sems):
    c = lax.axis_index("c")
    s = lax.axis_index("s")
    b0 = (s * NC + c) * BCOL

    xbufs = (xb0, xb1)
    ybufs = (yb0, yb1)

    def slabs(i, h):
        r = N - i
        src_x = inp.at[i, pl.ds(i, r), pl.ds(h * 8, 8), pl.ds(b0, BCOL)]
        src_y = inp.at[pl.ds(i, r), i, pl.ds(h * 8, 8), pl.ds(b0, BCOL)]
        return src_x, src_y

    def start(i, h):
        src_x, src_y = slabs(i, h)
        r = N - i
        pltpu.async_copy(src_x, xbufs[h].at[pl.ds(0, r)], sems[h])
        pltpu.async_copy(src_y, ybufs[h].at[pl.ds(0, r)], sems[h])

    def wait(i, h):
        src_x, src_y = slabs(i, h)
        r = N - i
        pltpu.make_async_copy(src_x, xbufs[h].at[pl.ds(0, r)], sems[h]).wait()
        pltpu.make_async_copy(src_y, ybufs[h].at[pl.ds(0, r)], sems[h]).wait()

    start(IT, 0)
    start(IT, 1)

    for i in range(IT, N):
        r = N - i
        for h in range(2):
            wait(i, h)
            xb = xbufs[h]
            yb = ybufs[h]

            if h == 0:
                @pl.loop(0, r)
                def _pairs0(j):
                    @pl.loop(0, LG, unroll=2)
                    def _lanes0(lg):
                        bs = lg * 16
                        v = xb[j, 0, pl.ds(bs, 16)] * yb[j, 0, pl.ds(bs, 16)]
                        for dd in range(1, 8):
                            v += xb[j, dd, pl.ds(bs, 16)] * yb[j, dd, pl.ds(bs, 16)]
                        out_v[_KB[i] + j, pl.ds(bs, 16)] = v
            else:
                @pl.loop(0, r)
                def _pairs1(j):
                    @pl.loop(0, LG, unroll=2)
                    def _lanes1(lg):
                        bs = lg * 16
                        v = out_v[_KB[i] + j, pl.ds(bs, 16)]
                        for dd in range(8):
                            v += xb[j, dd, pl.ds(bs, 16)] * yb[j, dd, pl.ds(bs, 16)]
                        out_v[_KB[i] + j, pl.ds(bs, 16)] = v

            if i + 1 < N:
                start(i + 1, h)

    pltpu.sync_copy(out_v, out.at[pl.ds(0, NSCP), pl.ds(b0, BCOL)])


def _tc_body(x_ref, y_ref, o_ref):
    o_ref[0, pl.ds(0, N)] = jnp.sum(x_ref[0] * y_ref[:, 0], axis=1)


@jax.jit
def kernel(input):
    inp = jnp.transpose(input, (1, 2, 0)).reshape(N, N, D, BATCH)
    mesh = plsc.VectorSubcoreMesh(
        core_axis_name="c", subcore_axis_name="s",
        num_cores=NC, num_subcores=NS)
    f = pl.kernel(
        _body,
        out_type=jax.ShapeDtypeStruct((NSCP, BATCH), jnp.float32),
        mesh=mesh,
        compiler_params=pltpu.CompilerParams(needs_layout_passes=False),
        scratch_types=[
            pltpu.VMEM((N - IT, 8, BCOL), jnp.float32),
            pltpu.VMEM((N - IT, 8, BCOL), jnp.float32),
            pltpu.VMEM((N - IT, 8, BCOL), jnp.float32),
            pltpu.VMEM((N - IT, 8, BCOL), jnp.float32),
            pltpu.VMEM((NSCP, BCOL), jnp.float32),
            (pltpu.SemaphoreType.DMA, pltpu.SemaphoreType.DMA),
        ],
    )
    sc_out = f(inp)
    tc = pl.pallas_call(
        _tc_body,
        grid=(IT, BATCH // BCT),
        in_specs=[
            pl.BlockSpec((1, N, D, BCT), lambda ib, cb: (ib, 0, 0, cb)),
            pl.BlockSpec((N, 1, D, BCT), lambda ib, cb: (0, ib, 0, cb)),
        ],
        out_specs=pl.BlockSpec((1, 32, BCT), lambda ib, cb: (ib, 0, cb)),
        out_shape=jax.ShapeDtypeStruct((IT, 32, BATCH), jnp.float32),
    )
    tc_out = tc(inp, inp)
    a = jnp.take(tc_out.reshape(IT * 32, BATCH), jnp.asarray(_TC_ROWS), axis=0,
                 mode="clip")
    return jnp.concatenate([a, sc_out[:NSC]], axis=0).T


# speedup vs baseline: 6.5630x; 1.0007x over previous
import numpy as np
import jax
import jax.numpy as jnp
from jax import lax
from jax.experimental import pallas as pl
from jax.experimental.pallas import tpu as pltpu
from jax.experimental.pallas import tpu_sc as plsc

N = 26
D = 16
BATCH = 4096
NPAIR = N * (N + 1) // 2
NC, NS = 2, 16
NW = NC * NS
BCOL = BATCH // NW
LG = BCOL // 16

IT = 9
NSC = sum(N - i for i in range(IT, N))
NSCP = 160
_KB = {}
_k = 0
for _i in range(IT, N):
    _KB[_i] = _k
    _k += N - _i
BCT = 1024

_TC_ROWS = np.asarray(
    [i * 32 + j for i in range(IT) for j in range(i, N)], np.int32)


def _body(inp, out, xb0, xb1, yb0, yb1, out_v, sems):
    c = lax.axis_index("c")
    s = lax.axis_index("s")
    b0 = (s * NC + c) * BCOL

    xbufs = (xb0, xb1)
    ybufs = (yb0, yb1)

    def slabs(i, h):
        r = N - i
        src_x = inp.at[i, pl.ds(i, r), pl.ds(h * 8, 8), pl.ds(b0, BCOL)]
        src_y = inp.at[pl.ds(i, r), i, pl.ds(h * 8, 8), pl.ds(b0, BCOL)]
        return src_x, src_y

    def start(i, h):
        src_x, src_y = slabs(i, h)
        r = N - i
        pltpu.async_copy(src_x, xbufs[h].at[pl.ds(0, r)], sems[h])
        pltpu.async_copy(src_y, ybufs[h].at[pl.ds(0, r)], sems[h])

    def wait(i, h):
        src_x, src_y = slabs(i, h)
        r = N - i
        pltpu.make_async_copy(src_x, xbufs[h].at[pl.ds(0, r)], sems[h]).wait()
        pltpu.make_async_copy(src_y, ybufs[h].at[pl.ds(0, r)], sems[h]).wait()

    start(IT, 0)
    start(IT, 1)

    for i in range(IT, N):
        r = N - i
        for h in range(2):
            wait(i, h)
            xb = xbufs[h]
            yb = ybufs[h]

            if h == 0:
                @pl.loop(0, r)
                def _pairs0(j):
                    @pl.loop(0, LG, unroll=2)
                    def _lanes0(lg):
                        bs = lg * 16
                        v = xb[j, 0, pl.ds(bs, 16)] * yb[j, 0, pl.ds(bs, 16)]
                        for dd in range(1, 8):
                            v += xb[j, dd, pl.ds(bs, 16)] * yb[j, dd, pl.ds(bs, 16)]
                        out_v[_KB[i] + j, pl.ds(bs, 16)] = v
            else:
                @pl.loop(0, r)
                def _pairs1(j):
                    @pl.loop(0, LG, unroll=2)
                    def _lanes1(lg):
                        bs = lg * 16
                        v = out_v[_KB[i] + j, pl.ds(bs, 16)]
                        for dd in range(8):
                            v += xb[j, dd, pl.ds(bs, 16)] * yb[j, dd, pl.ds(bs, 16)]
                        out_v[_KB[i] + j, pl.ds(bs, 16)] = v

            if i + 1 < N:
                start(i + 1, h)

    pltpu.sync_copy(out_v, out.at[pl.ds(0, NSCP), pl.ds(b0, BCOL)])


def _tc_body(x_ref, y_ref, o_ref):
    o_ref[0, pl.ds(0, N)] = jnp.sum(x_ref[0] * y_ref[:, 0], axis=1)


@jax.jit
def kernel(input):
    inp = jnp.transpose(input, (1, 2, 0)).reshape(N, N, D, BATCH)
    mesh = plsc.VectorSubcoreMesh(
        core_axis_name="c", subcore_axis_name="s",
        num_cores=NC, num_subcores=NS)
    f = pl.kernel(
        _body,
        out_type=jax.ShapeDtypeStruct((NSCP, BATCH), jnp.float32),
        mesh=mesh,
        compiler_params=pltpu.CompilerParams(needs_layout_passes=False),
        scratch_types=[
            pltpu.VMEM((N - IT, 8, BCOL), jnp.float32),
            pltpu.VMEM((N - IT, 8, BCOL), jnp.float32),
            pltpu.VMEM((N - IT, 8, BCOL), jnp.float32),
            pltpu.VMEM((N - IT, 8, BCOL), jnp.float32),
            pltpu.VMEM((NSCP, BCOL), jnp.float32),
            (pltpu.SemaphoreType.DMA, pltpu.SemaphoreType.DMA),
        ],
    )
    sc_out = f(inp)
    tc = pl.pallas_call(
        _tc_body,
        grid=(IT, BATCH // BCT),
        in_specs=[
            pl.BlockSpec((1, N, D, BCT), lambda ib, cb: (ib, 0, 0, cb)),
            pl.BlockSpec((N, 1, D, BCT), lambda ib, cb: (0, ib, 0, cb)),
        ],
        out_specs=pl.BlockSpec((1, 32, BCT), lambda ib, cb: (ib, 0, cb)),
        out_shape=jax.ShapeDtypeStruct((IT, 32, BATCH), jnp.float32),
    )
    tc_out = tc(inp, inp)
    a = jnp.take(tc_out.reshape(IT * 32, BATCH), jnp.asarray(_TC_ROWS), axis=0,
                 mode="clip")
    return jnp.concatenate([a, sc_out[:NSC]], axis=0).T
